# Initial kernel scaffold; baseline (speedup 1.0000x reference)
import jax
import jax.numpy as jnp
from jax.experimental import pallas as pl


def kernel(inputs, labels, sample_ids, weight):
    raise NotImplementedError("write your pallas kernel here")


# speedup vs baseline: 6.3829x; 6.3829x over previous
import functools

import jax
import jax.numpy as jnp
from jax import lax
from jax.experimental import pallas as pl
from jax.experimental.pallas import tpu as pltpu
from jax.experimental.pallas import tpu_sc as plsc

_NC = 2
_NS = 16
_NW = _NC * _NS


def _gather_body(b_per_w, table, idx, out, idx_v, rows_v, sem):
    wid = lax.axis_index("s") * _NC + lax.axis_index("c")
    base = wid * b_per_w
    pltpu.sync_copy(idx.at[pl.ds(base, b_per_w)], idx_v)
    pltpu.async_copy(table.at[idx_v], rows_v, sem).wait()
    pltpu.sync_copy(rows_v, out.at[pl.ds(base, b_per_w)])


def _sc_gather(table, idx):
    n, d = idx.shape[0], table.shape[1]
    b_per_w = n // _NW
    mesh = plsc.VectorSubcoreMesh(core_axis_name="c", subcore_axis_name="s")
    return pl.kernel(
        functools.partial(_gather_body, b_per_w),
        out_type=jax.ShapeDtypeStruct((n, d), table.dtype),
        mesh=mesh,
        scratch_types=[
            pltpu.VMEM((b_per_w,), jnp.int32),
            pltpu.VMEM((b_per_w, d), table.dtype),
            pltpu.SemaphoreType.DMA,
        ],
    )(table, idx)


def _dense_body(x_ref, tw_ref, sw_ref, out_ref):
    x = x_ref[...]
    sw = sw_ref[...]
    tw = tw_ref[...]
    x2 = jnp.sum(x * x, axis=1, keepdims=True)
    sw2 = jnp.sum(sw * sw, axis=1, keepdims=True)
    g = lax.dot_general(x, sw, (((1,), (1,)), ((), ())),
                        preferred_element_type=jnp.float32)
    d2 = x2 + jnp.transpose(sw2) - 2.0 * g
    dist = jnp.sqrt(jnp.maximum(d2, 0.0))
    s = jnp.sum(jnp.exp(dist), axis=1, keepdims=True)
    diff = x - tw
    td2 = jnp.sum(diff * diff, axis=1, keepdims=True)
    out_ref[...] = jnp.sqrt(td2) - jnp.log(s)


def _dense(inputs, tw, sw):
    b = inputs.shape[0]
    return pl.pallas_call(
        _dense_body,
        out_shape=jax.ShapeDtypeStruct((b, 1), jnp.float32),
    )(inputs, tw, sw)


def kernel(inputs, labels, sample_ids, weight):
    b = labels.shape[0]
    idx = jnp.concatenate(
        [labels.astype(jnp.int32), sample_ids.astype(jnp.int32)])
    rows = _sc_gather(weight, idx)
    tw = rows[:b]
    sw = rows[b:]
    out = _dense(inputs, tw, sw)
    return out[:, 0]


# speedup vs baseline: 6.8705x; 1.0764x over previous
import functools

import jax
import jax.numpy as jnp
from jax import lax
from jax.experimental import pallas as pl
from jax.experimental.pallas import tpu as pltpu
from jax.experimental.pallas import tpu_sc as plsc

_NC = 2
_NS = 16
_NW = _NC * _NS


def _gather_body(b_per_w, b, table, labels, samples, out, idx_v, rows_v, sem):
    wid = lax.axis_index("s") * _NC + lax.axis_index("c")
    base = wid * b_per_w

    @pl.when(base < b)
    def _():
        pltpu.sync_copy(labels.at[pl.ds(base, b_per_w)], idx_v)

    @pl.when(base >= b)
    def _():
        pltpu.sync_copy(samples.at[pl.ds(base - b, b_per_w)], idx_v)

    pltpu.async_copy(table.at[idx_v], rows_v, sem).wait()
    pltpu.sync_copy(rows_v, out.at[pl.ds(base, b_per_w)])


def _sc_gather(table, labels, samples):
    b, s, d = labels.shape[0], samples.shape[0], table.shape[1]
    b_per_w = (b + s) // _NW
    mesh = plsc.VectorSubcoreMesh(core_axis_name="c", subcore_axis_name="s")
    return pl.kernel(
        functools.partial(_gather_body, b_per_w, b),
        out_type=jax.ShapeDtypeStruct((b + s, d), table.dtype),
        mesh=mesh,
        scratch_types=[
            pltpu.VMEM((b_per_w,), jnp.int32),
            pltpu.VMEM((b_per_w, d), table.dtype),
            pltpu.SemaphoreType.DMA,
        ],
    )(table, labels, samples)


def _dense_body(b, x_ref, rows_ref, out_ref):
    x = x_ref[...]
    tw = rows_ref[:b, :]
    sw = rows_ref[b:, :]
    x2 = jnp.sum(x * x, axis=1, keepdims=True)
    sw2 = jnp.sum(sw * sw, axis=1, keepdims=True)
    g = lax.dot_general(x, sw, (((1,), (1,)), ((), ())),
                        preferred_element_type=jnp.float32)
    d2 = x2 + jnp.transpose(sw2) - 2.0 * g
    dist = jnp.sqrt(jnp.maximum(d2, 0.0))
    s = jnp.sum(jnp.exp(dist), axis=1, keepdims=True)
    diff = x - tw
    td2 = jnp.sum(diff * diff, axis=1, keepdims=True)
    out_ref[...] = jnp.sqrt(td2) - jnp.log(s)


def _dense(inputs, rows):
    b = inputs.shape[0]
    return pl.pallas_call(
        functools.partial(_dense_body, b),
        out_shape=jax.ShapeDtypeStruct((b, 1), jnp.float32),
    )(inputs, rows)


def kernel(inputs, labels, sample_ids, weight):
    rows = _sc_gather(weight, labels.astype(jnp.int32),
                      sample_ids.astype(jnp.int32))
    out = _dense(inputs, rows)
    return out[:, 0]


# speedup vs baseline: 7.4326x; 1.0818x over previous
import functools

import jax
import jax.numpy as jnp
from jax import lax
from jax.experimental import pallas as pl
from jax.experimental.pallas import tpu as pltpu
from jax.experimental.pallas import tpu_sc as plsc

_NC = 2
_NS = 16
_NW = _NC * _NS


def _gather_body(b_per_w, b, table, labels, samples, out, idx_v, rows_v, sem):
    wid = lax.axis_index("s") * _NC + lax.axis_index("c")
    base = wid * b_per_w

    @pl.when(base < b)
    def _():
        pltpu.sync_copy(labels.at[pl.ds(base, b_per_w)], idx_v)

    @pl.when(base >= b)
    def _():
        pltpu.sync_copy(samples.at[pl.ds(base - b, b_per_w)], idx_v)

    pltpu.async_copy(table.at[idx_v], rows_v, sem).wait()
    pltpu.sync_copy(rows_v, out.at[pl.ds(base, b_per_w)])


def _sc_gather(table, labels, samples):
    b, s, d = labels.shape[0], samples.shape[0], table.shape[1]
    b_per_w = (b + s) // _NW
    mesh = plsc.VectorSubcoreMesh(core_axis_name="c", subcore_axis_name="s")
    return pl.kernel(
        functools.partial(_gather_body, b_per_w, b),
        out_type=jax.ShapeDtypeStruct((b + s, d), table.dtype),
        mesh=mesh,
        scratch_types=[
            pltpu.VMEM((b_per_w,), jnp.int32),
            pltpu.VMEM((b_per_w, d), table.dtype),
            pltpu.SemaphoreType.DMA,
        ],
    )(table, labels, samples)


def _dense_body(b, x_ref, rows_ref, out_ref):
    x = x_ref[...]
    tw = rows_ref[:b, :]
    sw = rows_ref[b:, :]
    x2 = jnp.sum(x * x, axis=1, keepdims=True)
    sw2 = jnp.sum(sw * sw, axis=1, keepdims=True)
    g = lax.dot_general(x, sw, (((1,), (1,)), ((), ())),
                        preferred_element_type=jnp.float32)
    d2 = x2 + jnp.transpose(sw2) - 2.0 * g
    dist = jnp.sqrt(jnp.maximum(d2, 0.0))
    s = jnp.sum(jnp.exp(dist), axis=1)
    diff = x - tw
    td2 = jnp.sum(diff * diff, axis=1)
    out_ref[...] = jnp.sqrt(td2) - jnp.log(s)


def _dense(inputs, rows):
    b = inputs.shape[0]
    return pl.pallas_call(
        functools.partial(_dense_body, b),
        out_shape=jax.ShapeDtypeStruct((b,), jnp.float32),
    )(inputs, rows)


def kernel(inputs, labels, sample_ids, weight):
    rows = _sc_gather(weight, labels.astype(jnp.int32),
                      sample_ids.astype(jnp.int32))
    return _dense(inputs, rows)


# speedup vs baseline: 7.5811x; 1.0200x over previous
import functools

import jax
import jax.numpy as jnp
from jax import lax
from jax.experimental import pallas as pl
from jax.experimental.pallas import tpu as pltpu
from jax.experimental.pallas import tpu_sc as plsc

_NC = 1
_NS = 16
_NW = _NC * _NS


def _gather_body(b_per_w, b, table, labels, samples, out, idx_v, rows_v, sem):
    wid = lax.axis_index("s") * _NC + lax.axis_index("c")
    base = wid * b_per_w

    @pl.when(base < b)
    def _():
        pltpu.sync_copy(labels.at[pl.ds(base, b_per_w)], idx_v)

    @pl.when(base >= b)
    def _():
        pltpu.sync_copy(samples.at[pl.ds(base - b, b_per_w)], idx_v)

    pltpu.async_copy(table.at[idx_v], rows_v, sem).wait()
    pltpu.sync_copy(rows_v, out.at[pl.ds(base, b_per_w)])


def _sc_gather(table, labels, samples):
    b, s, d = labels.shape[0], samples.shape[0], table.shape[1]
    b_per_w = (b + s) // _NW
    mesh = plsc.VectorSubcoreMesh(core_axis_name="c", subcore_axis_name="s",
                                  num_cores=_NC)
    return pl.kernel(
        functools.partial(_gather_body, b_per_w, b),
        out_type=jax.ShapeDtypeStruct((b + s, d), table.dtype),
        mesh=mesh,
        scratch_types=[
            pltpu.VMEM((b_per_w,), jnp.int32),
            pltpu.VMEM((b_per_w, d), table.dtype),
            pltpu.SemaphoreType.DMA,
        ],
    )(table, labels, samples)


def _dense_body(b, x_ref, rows_ref, out_ref):
    x = x_ref[...]
    tw = rows_ref[:b, :]
    sw = rows_ref[b:, :]
    x2 = jnp.sum(x * x, axis=1, keepdims=True)
    sw2 = jnp.sum(sw * sw, axis=1, keepdims=True)
    g = lax.dot_general(x, sw, (((1,), (1,)), ((), ())),
                        preferred_element_type=jnp.float32)
    d2 = x2 + jnp.transpose(sw2) - 2.0 * g
    dist = jnp.sqrt(jnp.maximum(d2, 0.0))
    s = jnp.sum(jnp.exp(dist), axis=1)
    diff = x - tw
    td2 = jnp.sum(diff * diff, axis=1)
    out_ref[...] = jnp.sqrt(td2) - jnp.log(s)


def _dense(inputs, rows):
    b = inputs.shape[0]
    return pl.pallas_call(
        functools.partial(_dense_body, b),
        out_shape=jax.ShapeDtypeStruct((b,), jnp.float32),
    )(inputs, rows)


def kernel(inputs, labels, sample_ids, weight):
    rows = _sc_gather(weight, labels.astype(jnp.int32),
                      sample_ids.astype(jnp.int32))
    return _dense(inputs, rows)
